# baseline (speedup 1.0000x reference)
import jax
import jax.numpy as jnp
from jax import lax
from jax.experimental import pallas as pl
from jax.experimental.pallas import tpu as pltpu
from jax.experimental.pallas import tpu_sc as plsc

VOCAB = 100000
EMB = 128
BATCH = 4096
SEQ = 50
HIDDEN = 64
OUT = 10

NUM_CORES = 2
NUM_SUBCORES = 16
NW = NUM_CORES * NUM_SUBCORES
ROWS_PER_W = BATCH // NW
LANES = 16
NCH = EMB // LANES
RCH = ROWS_PER_W // LANES


def _sc_body(x_hbm, table_hbm, out_hbm, x_v, idx_v, acc0, acc1, sem_x, sem):
  wid = lax.axis_index("s") * NUM_CORES + lax.axis_index("c")
  base = wid * ROWS_PER_W

  pltpu.sync_copy(x_hbm.at[wid], x_v)

  lane_scaled = lax.iota(jnp.int32, LANES) * SEQ

  def build_row(j):
    for rc in range(RCH):
      idx16 = lane_scaled + (rc * (LANES * SEQ) + j)
      vals = plsc.load_gather(x_v, [idx16])
      idx_v[j, pl.ds(rc * LANES, LANES)] = vals

  build_row(0)
  build_row(1)
  pltpu.async_copy(table_hbm.at[idx_v.at[0]], acc0, sem)
  pltpu.async_copy(table_hbm.at[idx_v.at[1]], acc1, sem)

  def fire_even(j2, _):
    j = 2 + 2 * j2
    build_row(j)
    pltpu.async_copy(table_hbm.at[idx_v.at[j]], acc0, sem, add=True)
    build_row(j + 1)
    pltpu.async_copy(table_hbm.at[idx_v.at[j + 1]], acc1, sem, add=True)
    return 0

  lax.fori_loop(0, (SEQ - 2) // 2, fire_even, 0)

  def drain(j, _):
    pltpu.make_async_copy(table_hbm.at[idx_v.at[0]], acc0, sem).wait()
    return 0

  lax.fori_loop(0, SEQ, drain, 0)

  def combine(r, _):
    for c in range(NCH):
      s = pl.ds(c * LANES, LANES)
      acc0[r, s] = acc0[r, s] + acc1[r, s]
    return 0

  lax.fori_loop(0, ROWS_PER_W, combine, 0)
  pltpu.sync_copy(acc0, out_hbm.at[pl.ds(base, ROWS_PER_W), :])


def _sc_gather_sum(x2, table):
  mesh = plsc.VectorSubcoreMesh(core_axis_name="c", subcore_axis_name="s")
  k = pl.kernel(
      _sc_body,
      out_type=jax.ShapeDtypeStruct((BATCH, EMB), jnp.float32),
      mesh=mesh,
      compiler_params=pltpu.CompilerParams(needs_layout_passes=False),
      scratch_types=[
          pltpu.VMEM((ROWS_PER_W * SEQ,), jnp.int32),
          pltpu.VMEM((SEQ, ROWS_PER_W), jnp.int32),
          pltpu.VMEM((ROWS_PER_W, EMB), jnp.float32),
          pltpu.VMEM((ROWS_PER_W, EMB), jnp.float32),
          pltpu.SemaphoreType.DMA,
          pltpu.SemaphoreType.DMA,
      ],
  )
  return k(x2, table)


def _mlp_body(sums_ref, len_ref, w1_ref, b1_ref, w2_ref, b2_ref, out_ref):
  s = sums_ref[...]
  inv = 1.0 / len_ref[...].astype(jnp.float32)
  rep = s * inv
  h = lax.dot_general(rep, w1_ref[...], (((1,), (1,)), ((), ())),
                      preferred_element_type=jnp.float32)
  h = jnp.maximum(h + b1_ref[...], 0.0)
  o = lax.dot_general(h, w2_ref[...], (((1,), (1,)), ((), ())),
                      preferred_element_type=jnp.float32)
  out_ref[...] = o + b2_ref[...]


def _tc_mlp(sums, lengths2, W1, b1, W2, b2):
  return pl.pallas_call(
      _mlp_body,
      out_shape=jax.ShapeDtypeStruct((BATCH, OUT), jnp.float32),
  )(sums, lengths2, W1, b1.reshape(1, HIDDEN), W2, b2.reshape(1, OUT))


def kernel(x, lengths, table, W1, b1, W2, b2):
  x2 = x.reshape(NW, ROWS_PER_W * SEQ)
  sums = _sc_gather_sum(x2, table)
  return _tc_mlp(sums, lengths.reshape(BATCH, 1), W1, b1, W2, b2)


# speedup vs baseline: 1.0371x; 1.0371x over previous
import jax
import jax.numpy as jnp
from jax import lax
from jax.experimental import pallas as pl
from jax.experimental.pallas import tpu as pltpu
from jax.experimental.pallas import tpu_sc as plsc

VOCAB = 100000
EMB = 128
BATCH = 4096
SEQ = 50
HIDDEN = 64
OUT = 10

NUM_CORES = 2
NUM_SUBCORES = 16
NW = NUM_CORES * NUM_SUBCORES
ROWS_PER_W = BATCH // NW
LANES = 16
NCH = EMB // LANES


def _sc_body(xt_hbm, table_hbm, out_hbm, idx_v, acc, sem_x, sem):
  wid = lax.axis_index("s") * NUM_CORES + lax.axis_index("c")
  base = wid * ROWS_PER_W

  cp = pltpu.async_copy(xt_hbm.at[wid], idx_v, sem_x)

  zeros = jnp.zeros((LANES,), jnp.float32)

  def zero_body(r, _):
    for c in range(NCH):
      acc[r, pl.ds(c * LANES, LANES)] = zeros
    return 0

  lax.fori_loop(0, ROWS_PER_W, zero_body, 0)
  cp.wait()

  def fire(j, _):
    pltpu.async_copy(table_hbm.at[idx_v.at[j]], acc, sem, add=True)
    return 0

  lax.fori_loop(0, SEQ, fire, 0)

  def drain(j, _):
    pltpu.make_async_copy(table_hbm.at[idx_v.at[0]], acc, sem).wait()
    return 0

  lax.fori_loop(0, SEQ, drain, 0)

  pltpu.sync_copy(acc, out_hbm.at[pl.ds(base, ROWS_PER_W), :])


def _sc_gather_sum(xt, table):
  mesh = plsc.VectorSubcoreMesh(core_axis_name="c", subcore_axis_name="s")
  k = pl.kernel(
      _sc_body,
      out_type=jax.ShapeDtypeStruct((BATCH, EMB), jnp.float32),
      mesh=mesh,
      scratch_types=[
          pltpu.VMEM((SEQ, ROWS_PER_W), jnp.int32),
          pltpu.VMEM((ROWS_PER_W, EMB), jnp.float32),
          pltpu.SemaphoreType.DMA,
          pltpu.SemaphoreType.DMA,
      ],
  )
  return k(xt, table)


def _mlp_body(sums_ref, len_ref, w1_ref, b1_ref, w2_ref, b2_ref, out_ref):
  s = sums_ref[...]
  inv = 1.0 / len_ref[...].astype(jnp.float32)
  rep = s * inv
  h = lax.dot_general(rep, w1_ref[...], (((1,), (1,)), ((), ())),
                      preferred_element_type=jnp.float32)
  h = jnp.maximum(h + b1_ref[...], 0.0)
  o = lax.dot_general(h, w2_ref[...], (((1,), (1,)), ((), ())),
                      preferred_element_type=jnp.float32)
  out_ref[...] = o + b2_ref[...]


def _tc_mlp(sums, lengths2, W1, b1, W2, b2):
  return pl.pallas_call(
      _mlp_body,
      out_shape=jax.ShapeDtypeStruct((BATCH, OUT), jnp.float32),
  )(sums, lengths2, W1, b1.reshape(1, HIDDEN), W2, b2.reshape(1, OUT))


def kernel(x, lengths, table, W1, b1, W2, b2):
  xt = x.reshape(NW, ROWS_PER_W, SEQ).transpose(0, 2, 1)
  sums = _sc_gather_sum(xt, table)
  return _tc_mlp(sums, lengths.reshape(BATCH, 1), W1, b1, W2, b2)


# speedup vs baseline: 1.0379x; 1.0007x over previous
import jax
import jax.numpy as jnp
from jax import lax
from jax.experimental import pallas as pl
from jax.experimental.pallas import tpu as pltpu
from jax.experimental.pallas import tpu_sc as plsc

VOCAB = 100000
EMB = 128
BATCH = 4096
SEQ = 50
HIDDEN = 64
OUT = 10

NUM_CORES = 2
NUM_SUBCORES = 16
NW = NUM_CORES * NUM_SUBCORES
ROWS_PER_W = BATCH // NW
LANES = 16
NCH = EMB // LANES


def _sc_body(xt_hbm, table_hbm, out_hbm, idx_v, acc, sem_x, sem):
  wid = lax.axis_index("s") * NUM_CORES + lax.axis_index("c")
  base = wid * ROWS_PER_W

  cp = pltpu.async_copy(xt_hbm.at[wid], idx_v, sem_x)

  zeros = jnp.zeros((LANES,), jnp.float32)

  def zero_body(r, _):
    for c in range(NCH):
      acc[r, pl.ds(c * LANES, LANES)] = zeros
    return 0

  lax.fori_loop(0, ROWS_PER_W, zero_body, 0)
  cp.wait()

  H = ROWS_PER_W // 2

  def fire(j, _):
    pltpu.async_copy(table_hbm.at[idx_v.at[j, pl.ds(0, H)]],
                     acc.at[pl.ds(0, H), :], sem, add=True)
    pltpu.async_copy(table_hbm.at[idx_v.at[j, pl.ds(H, H)]],
                     acc.at[pl.ds(H, H), :], sem, add=True)
    return 0

  lax.fori_loop(0, SEQ, fire, 0)

  def drain(j, _):
    pltpu.make_async_copy(table_hbm.at[idx_v.at[0, pl.ds(0, H)]],
                          acc.at[pl.ds(0, H), :], sem).wait()
    return 0

  lax.fori_loop(0, 2 * SEQ, drain, 0)

  pltpu.sync_copy(acc, out_hbm.at[pl.ds(base, ROWS_PER_W), :])


def _sc_gather_sum(xt, table):
  mesh = plsc.VectorSubcoreMesh(core_axis_name="c", subcore_axis_name="s")
  k = pl.kernel(
      _sc_body,
      out_type=jax.ShapeDtypeStruct((BATCH, EMB), jnp.float32),
      mesh=mesh,
      scratch_types=[
          pltpu.VMEM((SEQ, ROWS_PER_W), jnp.int32),
          pltpu.VMEM((ROWS_PER_W, EMB), jnp.float32),
          pltpu.SemaphoreType.DMA,
          pltpu.SemaphoreType.DMA,
      ],
  )
  return k(xt, table)


def _mlp_body(sums_ref, len_ref, w1_ref, b1_ref, w2_ref, b2_ref, out_ref):
  s = sums_ref[...]
  inv = 1.0 / len_ref[...].astype(jnp.float32)
  rep = s * inv
  h = lax.dot_general(rep, w1_ref[...], (((1,), (1,)), ((), ())),
                      preferred_element_type=jnp.float32)
  h = jnp.maximum(h + b1_ref[...], 0.0)
  o = lax.dot_general(h, w2_ref[...], (((1,), (1,)), ((), ())),
                      preferred_element_type=jnp.float32)
  out_ref[...] = o + b2_ref[...]


def _tc_mlp(sums, lengths2, W1, b1, W2, b2):
  return pl.pallas_call(
      _mlp_body,
      out_shape=jax.ShapeDtypeStruct((BATCH, OUT), jnp.float32),
  )(sums, lengths2, W1, b1.reshape(1, HIDDEN), W2, b2.reshape(1, OUT))


def kernel(x, lengths, table, W1, b1, W2, b2):
  xt = x.reshape(NW, ROWS_PER_W, SEQ).transpose(0, 2, 1)
  sums = _sc_gather_sum(xt, table)
  return _tc_mlp(sums, lengths.reshape(BATCH, 1), W1, b1, W2, b2)
